# Initial kernel scaffold; baseline (speedup 1.0000x reference)
import jax
import jax.numpy as jnp
from jax.experimental import pallas as pl


def kernel(x, edge_index, eta, phi, W_s, b_s, W1, b1, g1, be1, W2, b2, g2, be2):
    raise NotImplementedError("write your pallas kernel here")


# speedup vs baseline: 5.2369x; 5.2369x over previous
import functools

import jax
import jax.numpy as jnp
from jax import lax
from jax.experimental import pallas as pl
from jax.experimental.pallas import tpu as pltpu
from jax.experimental.pallas import tpu_sc as plsc

F32 = jnp.float32


def _dot_t(a, b):
    return lax.dot_general(a, b, (((1,), (1,)), ((), ())),
                           preferred_element_type=F32)


def _proj_body(x_ref, ws_ref, bs_ref, s_ref, sq_ref):
    s = _dot_t(x_ref[...], ws_ref[...]) + bs_ref[...]
    s_ref[...] = s
    sq_ref[...] = jnp.sum(s * s, axis=1, keepdims=True)


def _proj(x, W_s, b_s, br):
    n, d_in = x.shape
    kd = W_s.shape[0]
    return pl.pallas_call(
        _proj_body,
        grid=(n // br,),
        in_specs=[
            pl.BlockSpec((br, d_in), lambda i: (i, 0)),
            pl.BlockSpec((kd, d_in), lambda i: (0, 0)),
            pl.BlockSpec((1, kd), lambda i: (0, 0)),
        ],
        out_specs=[
            pl.BlockSpec((br, kd), lambda i: (i, 0)),
            pl.BlockSpec((br, 1), lambda i: (i, 0)),
        ],
        out_shape=[
            jax.ShapeDtypeStruct((n, kd), F32),
            jax.ShapeDtypeStruct((n, 1), F32),
        ],
    )(x, W_s, b_s.reshape(1, -1))


def _topk_body(k, s_r_ref, sq_r_ref, s_all_ref, sq_c_ref, nbr_ref):
    s_r = s_r_ref[...]
    br = s_r.shape[0]
    n = s_all_ref.shape[0]
    d = (sq_r_ref[...] + sq_c_ref[...]) - 2.0 * _dot_t(s_r, s_all_ref[...])
    iota = lax.broadcasted_iota(jnp.int32, (br, n), 1)
    big = jnp.float32(jnp.inf)
    cols = []
    for _ in range(k):
        m = jnp.min(d, axis=1, keepdims=True)
        idx = jnp.min(jnp.where(d == m, iota, n), axis=1,
                      keepdims=True)
        cols.append(idx)
        d = jnp.where(iota == idx, big, d)
    nbr_ref[...] = jnp.concatenate(cols, axis=1)


def _topk(s, sq, k, br):
    n, kd = s.shape
    return pl.pallas_call(
        functools.partial(_topk_body, k),
        grid=(n // br,),
        in_specs=[
            pl.BlockSpec((br, kd), lambda i: (i, 0)),
            pl.BlockSpec((br, 1), lambda i: (i, 0)),
            pl.BlockSpec((n, kd), lambda i: (0, 0)),
            pl.BlockSpec((1, n), lambda i: (0, 0)),
        ],
        out_specs=pl.BlockSpec((br, k), lambda i: (i, 0)),
        out_shape=jax.ShapeDtypeStruct((n, k), jnp.int32),
    )(s, sq, s, sq.reshape(1, n))


def _gather_rows(table, idx):
    e = idx.shape[0]
    d = table.shape[1]
    info = plsc.get_sparse_core_info()
    nc, ns = info.num_cores, info.num_subcores
    nw = nc * ns
    e_per_w = e // nw
    ch = 200
    n_ch = e_per_w // ch
    mesh = plsc.VectorSubcoreMesh(core_axis_name="c", subcore_axis_name="s")

    @functools.partial(
        pl.kernel,
        mesh=mesh,
        out_type=jax.ShapeDtypeStruct((e, d), F32),
        scratch_types=[
            pltpu.VMEM((e_per_w,), jnp.int32),
            pltpu.VMEM((ch, d), F32),
            pltpu.SemaphoreType.DMA,
        ],
    )
    def gk(tbl_hbm, idx_hbm, out_hbm, idx_v, buf, sem):
        wid = lax.axis_index("s") * nc + lax.axis_index("c")
        base = wid * e_per_w
        pltpu.sync_copy(idx_hbm.at[pl.ds(base, e_per_w)], idx_v)

        def body(ci, carry):
            off = ci * ch
            pltpu.async_copy(
                tbl_hbm.at[idx_v.at[pl.ds(off, ch)]], buf, sem).wait()
            pltpu.sync_copy(buf, out_hbm.at[pl.ds(base + off, ch)])
            return carry

        lax.fori_loop(0, n_ch, body, 0, unroll=False)

    return gk(table, idx)


def _ln_relu(h, gamma, beta):
    mu = jnp.mean(h, axis=1, keepdims=True)
    r = h - mu
    var = jnp.mean(r * r, axis=1, keepdims=True)
    h = r / jnp.sqrt(var + 1e-5) * gamma + beta
    return jnp.maximum(h, 0.0)


def _mlp_body(k, xg_ref, x_ref, w1_ref, b1_ref, w2_ref, b2_ref, g1_ref,
              be1_ref, g2_ref, be2_ref, out_ref):
    bn, d = x_ref.shape
    x_j = xg_ref[...].reshape(bn, k, d)
    x_i = x_ref[...][:, None, :]
    m = jnp.concatenate(
        [jnp.broadcast_to(x_i, x_j.shape), x_j - x_i], axis=-1)
    m = m.reshape(bn * k, 2 * d)
    h = _dot_t(m, w1_ref[...]) + b1_ref[...]
    h = _ln_relu(h, g1_ref[...], be1_ref[...])
    h = _dot_t(h, w2_ref[...]) + b2_ref[...]
    h = _ln_relu(h, g2_ref[...], be2_ref[...])
    h = h.reshape(bn, k, d)
    acc = h[:, 0, :]
    for j in range(1, k):
        acc = acc + h[:, j, :]
    out_ref[...] = acc * (1.0 / k)


def _mlp(xg, x, W1, b1, W2, b2, g1, be1, g2, be2, k, bn):
    n, d = x.shape
    row2d = lambda v: v.reshape(1, -1)
    return pl.pallas_call(
        functools.partial(_mlp_body, k),
        grid=(n // bn,),
        in_specs=[
            pl.BlockSpec((bn * k, d), lambda i: (i, 0)),
            pl.BlockSpec((bn, d), lambda i: (i, 0)),
            pl.BlockSpec((d, 2 * d), lambda i: (0, 0)),
            pl.BlockSpec((1, d), lambda i: (0, 0)),
            pl.BlockSpec((d, d), lambda i: (0, 0)),
        ] + [pl.BlockSpec((1, d), lambda i: (0, 0))] * 5,
        out_specs=pl.BlockSpec((bn, d), lambda i: (i, 0)),
        out_shape=jax.ShapeDtypeStruct((n, d), F32),
    )(xg, x, W1, row2d(b1), W2, row2d(b2), row2d(g1), row2d(be1), row2d(g2),
      row2d(be2))


def kernel(x, edge_index, eta, phi, W_s, b_s, W1, b1, g1, be1, W2, b2, g2,
           be2):
    del edge_index, eta, phi
    k = 16
    br = 400
    s, sq = _proj(x, W_s, b_s, br)
    nbr = _topk(s, sq, k, br)
    xg = _gather_rows(x, nbr.reshape(-1))
    return _mlp(xg, x, W1, b1, W2, b2, g1, be1, g2, be2, k, br)
